# baseline (speedup 1.0000x reference)
import functools

import jax
import jax.numpy as jnp
from jax import lax
from jax.experimental import pallas as pl
from jax.experimental.pallas import tpu as pltpu
from jax.experimental.pallas import tpu_sc as plsc

_BOOST = 3.0
_CHUNK = 512


@functools.lru_cache(maxsize=None)
def _build(B, V, D, nc, ns):
    NW = nc * ns
    assert B % NW == 0
    bpw = B // NW
    assert bpw % _CHUNK == 0 and (bpw // _CHUNK) % 2 == 0
    nch = bpw // _CHUNK
    lanes = 16
    mesh = plsc.VectorSubcoreMesh(
        core_axis_name="c", subcore_axis_name="s", num_cores=nc, num_subcores=ns
    )

    @functools.partial(
        pl.kernel,
        out_type=jax.ShapeDtypeStruct((B, D), jnp.float32),
        mesh=mesh,
        scratch_types=[
            pltpu.VMEM((bpw,), jnp.int32),
            pltpu.VMEM((_CHUNK, D), jnp.float32),
            pltpu.VMEM((_CHUNK, D), jnp.float32),
            pltpu.SemaphoreType.DMA,
            pltpu.SemaphoreType.DMA,
        ],
        compiler_params=pltpu.CompilerParams(use_tc_tiling_on_sc=False),
    )
    def k(x_hbm, w_hbm, out_hbm, idx_v, rows0, rows1, sem0, sem1):
        wid = lax.axis_index("s") * nc + lax.axis_index("c")
        base = wid * bpw
        pltpu.sync_copy(x_hbm.at[pl.ds(base, bpw)], idx_v)

        def gather_start(g, buf, sem):
            return pltpu.async_copy(
                w_hbm.at[idx_v.at[pl.ds(g * _CHUNK, _CHUNK)]], buf, sem
            )

        def gather_wait(buf, sem):
            pltpu.make_async_copy(w_hbm.at[idx_v.at[pl.ds(0, _CHUNK)]], buf, sem).wait()

        def scale(buf):
            @plsc.parallel_loop(0, _CHUNK, 1, unroll=8)
            def _(r):
                for c in range(D // lanes):
                    sl = (r, pl.ds(c * lanes, lanes))
                    buf[sl] = buf[sl] * _BOOST

        def flush(g, buf):
            pltpu.sync_copy(buf, out_hbm.at[pl.ds(base + g * _CHUNK, _CHUNK)])

        gather_start(0, rows0, sem0)
        gather_start(1, rows1, sem1)

        @pl.loop(0, nch, step=2)
        def _(h):
            gather_wait(rows0, sem0)
            scale(rows0)
            flush(h, rows0)

            @pl.when(h + 2 < nch)
            def _():
                gather_start(h + 2, rows0, sem0)

            gather_wait(rows1, sem1)
            scale(rows1)
            flush(h + 1, rows1)

            @pl.when(h + 3 < nch)
            def _():
                gather_start(h + 3, rows1, sem1)

    return k


def kernel(x, weight):
    V, D = weight.shape
    B = x.size
    idx = x.reshape(-1).astype(jnp.int32)
    info = plsc.get_sparse_core_info()
    fn = _build(B, V, D, info.num_cores, info.num_subcores)
    out = fn(idx, weight)
    return out.reshape(x.shape + (D,))
